# baseline (speedup 1.0000x reference)
import functools

import jax
import jax.numpy as jnp
import numpy as np
from jax import lax
from jax.experimental import pallas as pl
from jax.experimental.pallas import tpu as pltpu
from jax.experimental.pallas import tpu_sc as plsc

T, U, I = 8, 1024, 2048
B = 4096
TOP_K = 50
NW = 32
NQ = 4
BQ = B // NQ
PER_W = BQ // NW
CHUNK = 16
NCHUNK = PER_W // CHUNK

_R = 512
_G = B // _R


def _f32_key(np_val):
    b = np.array(np_val, np.float32).view(np.int32)
    return int(b) if b >= 0 else int(b ^ 0x7FFFFFFF)


_LO_KEY = _f32_key(-1.5)
_HI_KEY = _f32_key(1.5)


_TB = 512


def _tr_body(in_ref, out_ref):
    out_ref[...] = in_ref[0].T


def _transpose_qos(qos):
    return pl.pallas_call(
        _tr_body,
        grid=(T, I // _TB),
        in_specs=[pl.BlockSpec((1, U, _TB), lambda t, ib: (t, 0, ib))],
        out_specs=pl.BlockSpec((_TB, U), lambda t, ib: (t * (I // _TB) + ib, 0)),
        out_shape=jax.ShapeDtypeStruct((T * I, U), jnp.float32),
    )(qos)


def _sc_body(qt, usim, tid, iid, uid, out_col, out_sim,
             tball, iball, uball, rall,
             colb0, colb1, simb0, simb1,
             sgc0, sgc1, sgs0, sgs1, swc0, swc1, sws0, sws1):
    cid = lax.axis_index("c")
    sid = lax.axis_index("s")
    wid = sid * 2 + cid
    w0 = wid * PER_W

    pltpu.sync_copy(tid.at[pl.ds(w0, PER_W)], tball)
    pltpu.sync_copy(iid.at[pl.ds(w0, PER_W)], iball)
    pltpu.sync_copy(uid.at[pl.ds(w0, PER_W)], uball)
    for k in range(PER_W // 16):
        sl = pl.ds(k * 16, 16)
        rall[sl] = tball[sl] * I + iball[sl]

    colb = (colb0, colb1)
    simb = (simb0, simb1)
    sgc = (sgc0, sgc1)
    sgs = (sgs0, sgs1)
    swc = (swc0, swc1)
    sws = (sws0, sws1)

    gc = [None, None]
    gs = [None, None]
    wc = [None, None]
    ws = [None, None]

    for ch in range(NCHUNK):
        s = ch % 2
        if ch >= 2:
            wc[s].wait()
            ws[s].wait()
        isl = pl.ds(ch * CHUNK, CHUNK)
        gc[s] = pltpu.async_copy(qt.at[rall.at[isl]], colb[s], sgc[s])
        gs[s] = pltpu.async_copy(usim.at[uball.at[isl]], simb[s], sgs[s])
        if ch >= 1:
            p = (ch - 1) % 2
            gc[p].wait()
            gs[p].wait()
            b0 = w0 + (ch - 1) * CHUNK
            wc[p] = pltpu.async_copy(colb[p], out_col.at[pl.ds(b0, CHUNK)], swc[p])
            ws[p] = pltpu.async_copy(simb[p], out_sim.at[pl.ds(b0, CHUNK)], sws[p])

    last = NCHUNK - 1
    s = last % 2
    gc[s].wait()
    gs[s].wait()
    b0 = w0 + last * CHUNK
    wc[s] = pltpu.async_copy(colb[s], out_col.at[pl.ds(b0, CHUNK)], swc[s])
    ws[s] = pltpu.async_copy(simb[s], out_sim.at[pl.ds(b0, CHUNK)], sws[s])
    wc[0].wait()
    ws[0].wait()
    wc[1].wait()
    ws[1].wait()


@functools.cache
def _get_sc_gather():
    return pl.kernel(
        _sc_body,
        mesh=plsc.VectorSubcoreMesh(core_axis_name="c", subcore_axis_name="s"),
        out_type=[
            jax.ShapeDtypeStruct((BQ, U), jnp.float32),
            jax.ShapeDtypeStruct((BQ, U), jnp.float32),
        ],
        scratch_types=[
            pltpu.VMEM((PER_W,), jnp.int32),
            pltpu.VMEM((PER_W,), jnp.int32),
            pltpu.VMEM((PER_W,), jnp.int32),
            pltpu.VMEM((PER_W,), jnp.int32),
            pltpu.VMEM((CHUNK, U), jnp.float32),
            pltpu.VMEM((CHUNK, U), jnp.float32),
            pltpu.VMEM((CHUNK, U), jnp.float32),
            pltpu.VMEM((CHUNK, U), jnp.float32),
            pltpu.SemaphoreType.DMA,
            pltpu.SemaphoreType.DMA,
            pltpu.SemaphoreType.DMA,
            pltpu.SemaphoreType.DMA,
            pltpu.SemaphoreType.DMA,
            pltpu.SemaphoreType.DMA,
            pltpu.SemaphoreType.DMA,
            pltpu.SemaphoreType.DMA,
        ],
    )


def _tc_body(tid_ref, uid_ref, col_ref, sim_ref, uavg_ref, out_ref):
    col = col_ref[...]
    sim = sim_ref[...]
    msim = jnp.where(col > 0.0, sim, 0.0)

    kbits = lax.bitcast_convert_type(msim, jnp.int32)
    key = jnp.where(kbits >= 0, kbits, kbits ^ jnp.int32(0x7FFFFFFF))

    tcol = tid_ref[:, 0:1]
    ucol = uid_ref[:, 0:1]

    avgv = jnp.zeros((_R, U), jnp.float32)
    for t in range(T):
        avgv = jnp.where(tcol == t, uavg_ref[t:t + 1, :], avgv)

    lo0 = jnp.full((_R, 1), _LO_KEY, jnp.int32)
    hi0 = jnp.full((_R, 1), _HI_KEY, jnp.int32)

    def _cnt(m):
        return jnp.sum((key > m).astype(jnp.int32), axis=1, keepdims=True)

    def quad(_, lh):
        lo, hi = lh
        s = hi - lo
        q = s >> 2
        m1 = lo + q
        m2 = lo + (s >> 1)
        m3 = hi - q
        g1 = _cnt(m1) >= TOP_K
        g2 = _cnt(m2) >= TOP_K
        g3 = _cnt(m3) >= TOP_K
        lo2 = jnp.where(g3, m3, jnp.where(g2, m2, jnp.where(g1, m1, lo)))
        hi2 = jnp.where(g3, hi, jnp.where(g2, m3, jnp.where(g1, m2, m1)))
        return lo2, hi2

    def bs(_, lh):
        lo, hi = lh
        mid = lo + ((hi - lo) >> 1)
        geq = _cnt(mid) >= TOP_K
        return jnp.where(geq, mid, lo), jnp.where(geq, hi, mid)

    lh = lax.fori_loop(0, 16, quad, (lo0, hi0))
    _, thk = lax.fori_loop(0, 2, bs, lh)

    tbits = jnp.where(thk >= 0, thk, thk ^ jnp.int32(0x7FFFFFFF))
    theta = lax.bitcast_convert_type(tbits, jnp.float32)

    strict = key > thk
    tie = key == thk
    d = col - avgv
    zero = jnp.zeros((), jnp.float32)
    s1 = jnp.sum(jnp.where(strict, msim, zero), axis=1, keepdims=True)
    c = jnp.sum(jnp.where(strict, 1.0, zero), axis=1, keepdims=True)
    nums = jnp.sum(jnp.where(strict, msim * d, zero), axis=1, keepdims=True)
    t2 = jnp.sum(jnp.where(tie, d, zero), axis=1, keepdims=True)
    tcnt = jnp.sum(jnp.where(tie, 1.0, zero), axis=1, keepdims=True)

    rem = jnp.float32(TOP_K) - c
    num = nums + theta * (rem / tcnt) * t2
    den = s1 + theta * rem

    lane = lax.broadcasted_iota(jnp.int32, (_R, U), 1)
    avg_u = jnp.sum(jnp.where(lane == ucol, avgv, zero), axis=1, keepdims=True)

    pred = avg_u + num / (den + 1e-8)
    out_ref[...] = jnp.broadcast_to(pred, (_R, 128))


def _tc_call(tid2, uid2, col, sim, uavg, interpret=False):
    return pl.pallas_call(
        _tc_body,
        grid=(col.shape[0] // _R,),
        in_specs=[
            pl.BlockSpec((_R, 128), lambda g: (g, 0)),
            pl.BlockSpec((_R, 128), lambda g: (g, 0)),
            pl.BlockSpec((_R, U), lambda g: (g, 0)),
            pl.BlockSpec((_R, U), lambda g: (g, 0)),
            pl.BlockSpec((T, U), lambda g: (0, 0)),
        ],
        out_specs=pl.BlockSpec((_R, 128), lambda g: (g, 0)),
        out_shape=jax.ShapeDtypeStruct((col.shape[0], 128), jnp.float32),
        interpret=interpret,
    )(tid2, uid2, col, sim, uavg)


def kernel(qos, user_avg, user_sim_agg, user_id, item_id, time_id):
    qt = _transpose_qos(qos)
    sc = _get_sc_gather()
    tid2 = jnp.broadcast_to(time_id[:, None], (B, 128))
    uid2 = jnp.broadcast_to(user_id[:, None], (B, 128))
    gathered = [sc(qt, user_sim_agg,
                   time_id[q * BQ:(q + 1) * BQ],
                   item_id[q * BQ:(q + 1) * BQ],
                   user_id[q * BQ:(q + 1) * BQ]) for q in range(NQ)]
    preds = [_tc_call(tid2[q * BQ:(q + 1) * BQ], uid2[q * BQ:(q + 1) * BQ],
                      col, sim, user_avg)
             for q, (col, sim) in enumerate(gathered)]
    return jnp.concatenate(preds, axis=0)[:, 0]


# speedup vs baseline: 1.1149x; 1.1149x over previous
import functools

import jax
import jax.numpy as jnp
import numpy as np
from jax import lax
from jax.experimental import pallas as pl
from jax.experimental.pallas import tpu as pltpu
from jax.experimental.pallas import tpu_sc as plsc

T, U, I = 8, 1024, 2048
B = 4096
TOP_K = 50
NW = 32
NQ = 4
BQ = B // NQ
PER_W = BQ // NW
CHUNK = 16
NCHUNK = PER_W // CHUNK

_R = 1024
_G = B // _R


def _f32_key(np_val):
    b = np.array(np_val, np.float32).view(np.int32)
    return int(b) if b >= 0 else int(b ^ 0x7FFFFFFF)


_LO_KEY = _f32_key(-1.5)
_HI_KEY = _f32_key(1.5)


_TB = 512


def _tr_body(in_ref, out_ref):
    out_ref[...] = in_ref[0].T


def _transpose_qos(qos):
    return pl.pallas_call(
        _tr_body,
        grid=(T, I // _TB),
        in_specs=[pl.BlockSpec((1, U, _TB), lambda t, ib: (t, 0, ib))],
        out_specs=pl.BlockSpec((_TB, U), lambda t, ib: (t * (I // _TB) + ib, 0)),
        out_shape=jax.ShapeDtypeStruct((T * I, U), jnp.float32),
    )(qos)


def _sc_body(qt, usim, tid, iid, uid, out_col, out_sim,
             tball, iball, uball, rall,
             colb0, colb1, simb0, simb1,
             sgc0, sgc1, sgs0, sgs1, swc0, swc1, sws0, sws1):
    cid = lax.axis_index("c")
    sid = lax.axis_index("s")
    wid = sid * 2 + cid
    w0 = wid * PER_W

    pltpu.sync_copy(tid.at[pl.ds(w0, PER_W)], tball)
    pltpu.sync_copy(iid.at[pl.ds(w0, PER_W)], iball)
    pltpu.sync_copy(uid.at[pl.ds(w0, PER_W)], uball)
    for k in range(PER_W // 16):
        sl = pl.ds(k * 16, 16)
        rall[sl] = tball[sl] * I + iball[sl]

    colb = (colb0, colb1)
    simb = (simb0, simb1)
    sgc = (sgc0, sgc1)
    sgs = (sgs0, sgs1)
    swc = (swc0, swc1)
    sws = (sws0, sws1)

    gc = [None, None]
    gs = [None, None]
    wc = [None, None]
    ws = [None, None]

    for ch in range(NCHUNK):
        s = ch % 2
        if ch >= 2:
            wc[s].wait()
            ws[s].wait()
        isl = pl.ds(ch * CHUNK, CHUNK)
        gc[s] = pltpu.async_copy(qt.at[rall.at[isl]], colb[s], sgc[s])
        gs[s] = pltpu.async_copy(usim.at[uball.at[isl]], simb[s], sgs[s])
        if ch >= 1:
            p = (ch - 1) % 2
            gc[p].wait()
            gs[p].wait()
            b0 = w0 + (ch - 1) * CHUNK
            wc[p] = pltpu.async_copy(colb[p], out_col.at[pl.ds(b0, CHUNK)], swc[p])
            ws[p] = pltpu.async_copy(simb[p], out_sim.at[pl.ds(b0, CHUNK)], sws[p])

    last = NCHUNK - 1
    s = last % 2
    gc[s].wait()
    gs[s].wait()
    b0 = w0 + last * CHUNK
    wc[s] = pltpu.async_copy(colb[s], out_col.at[pl.ds(b0, CHUNK)], swc[s])
    ws[s] = pltpu.async_copy(simb[s], out_sim.at[pl.ds(b0, CHUNK)], sws[s])
    wc[0].wait()
    ws[0].wait()
    wc[1].wait()
    ws[1].wait()


@functools.cache
def _get_sc_gather():
    return pl.kernel(
        _sc_body,
        mesh=plsc.VectorSubcoreMesh(core_axis_name="c", subcore_axis_name="s"),
        out_type=[
            jax.ShapeDtypeStruct((BQ, U), jnp.float32),
            jax.ShapeDtypeStruct((BQ, U), jnp.float32),
        ],
        scratch_types=[
            pltpu.VMEM((PER_W,), jnp.int32),
            pltpu.VMEM((PER_W,), jnp.int32),
            pltpu.VMEM((PER_W,), jnp.int32),
            pltpu.VMEM((PER_W,), jnp.int32),
            pltpu.VMEM((CHUNK, U), jnp.float32),
            pltpu.VMEM((CHUNK, U), jnp.float32),
            pltpu.VMEM((CHUNK, U), jnp.float32),
            pltpu.VMEM((CHUNK, U), jnp.float32),
            pltpu.SemaphoreType.DMA,
            pltpu.SemaphoreType.DMA,
            pltpu.SemaphoreType.DMA,
            pltpu.SemaphoreType.DMA,
            pltpu.SemaphoreType.DMA,
            pltpu.SemaphoreType.DMA,
            pltpu.SemaphoreType.DMA,
            pltpu.SemaphoreType.DMA,
        ],
    )


def _tc_body(tid_ref, uid_ref, col_ref, sim_ref, uavg_ref, out_ref):
    col = col_ref[...]
    sim = sim_ref[...]
    msim = jnp.where(col > 0.0, sim, 0.0)

    kbits = lax.bitcast_convert_type(msim, jnp.int32)
    key = jnp.where(kbits >= 0, kbits, kbits ^ jnp.int32(0x7FFFFFFF))

    tcol = tid_ref[:, 0:1]
    ucol = uid_ref[:, 0:1]

    avgv = jnp.zeros((_R, U), jnp.float32)
    for t in range(T):
        avgv = jnp.where(tcol == t, uavg_ref[t:t + 1, :], avgv)

    lo0 = jnp.full((_R, 1), _LO_KEY, jnp.int32)
    hi0 = jnp.full((_R, 1), _HI_KEY, jnp.int32)

    def _cnt(m):
        return jnp.sum((key > m).astype(jnp.int32), axis=1, keepdims=True)

    def bs(_, lh):
        lo, hi = lh
        mid = lo + ((hi - lo) >> 1)
        geq = _cnt(mid) >= TOP_K
        return jnp.where(geq, mid, lo), jnp.where(geq, hi, mid)

    _, thk = lax.fori_loop(0, 31, bs, (lo0, hi0))

    tbits = jnp.where(thk >= 0, thk, thk ^ jnp.int32(0x7FFFFFFF))
    theta = lax.bitcast_convert_type(tbits, jnp.float32)

    strict = key > thk
    tie = key == thk
    d = col - avgv
    zero = jnp.zeros((), jnp.float32)
    s1 = jnp.sum(jnp.where(strict, msim, zero), axis=1, keepdims=True)
    c = jnp.sum(jnp.where(strict, 1.0, zero), axis=1, keepdims=True)
    nums = jnp.sum(jnp.where(strict, msim * d, zero), axis=1, keepdims=True)
    t2 = jnp.sum(jnp.where(tie, d, zero), axis=1, keepdims=True)
    tcnt = jnp.sum(jnp.where(tie, 1.0, zero), axis=1, keepdims=True)

    rem = jnp.float32(TOP_K) - c
    num = nums + theta * (rem / tcnt) * t2
    den = s1 + theta * rem

    lane = lax.broadcasted_iota(jnp.int32, (_R, U), 1)
    avg_u = jnp.sum(jnp.where(lane == ucol, avgv, zero), axis=1, keepdims=True)

    pred = avg_u + num / (den + 1e-8)
    out_ref[...] = jnp.broadcast_to(pred, (_R, 128))


def _tc_call(tid2, uid2, col, sim, uavg, interpret=False):
    return pl.pallas_call(
        _tc_body,
        grid=(col.shape[0] // _R,),
        in_specs=[
            pl.BlockSpec((_R, 128), lambda g: (g, 0)),
            pl.BlockSpec((_R, 128), lambda g: (g, 0)),
            pl.BlockSpec((_R, U), lambda g: (g, 0)),
            pl.BlockSpec((_R, U), lambda g: (g, 0)),
            pl.BlockSpec((T, U), lambda g: (0, 0)),
        ],
        out_specs=pl.BlockSpec((_R, 128), lambda g: (g, 0)),
        out_shape=jax.ShapeDtypeStruct((col.shape[0], 128), jnp.float32),
        interpret=interpret,
    )(tid2, uid2, col, sim, uavg)


def kernel(qos, user_avg, user_sim_agg, user_id, item_id, time_id):
    qt = _transpose_qos(qos)
    sc = _get_sc_gather()
    tid2 = jnp.broadcast_to(time_id[:, None], (B, 128))
    uid2 = jnp.broadcast_to(user_id[:, None], (B, 128))
    gathered = [sc(qt, user_sim_agg,
                   time_id[q * BQ:(q + 1) * BQ],
                   item_id[q * BQ:(q + 1) * BQ],
                   user_id[q * BQ:(q + 1) * BQ]) for q in range(NQ)]
    preds = [_tc_call(tid2[q * BQ:(q + 1) * BQ], uid2[q * BQ:(q + 1) * BQ],
                      col, sim, user_avg)
             for q, (col, sim) in enumerate(gathered)]
    return jnp.concatenate(preds, axis=0)[:, 0]


# speedup vs baseline: 1.2155x; 1.0903x over previous
import functools

import jax
import jax.numpy as jnp
import numpy as np
from jax import lax
from jax.experimental import pallas as pl
from jax.experimental.pallas import tpu as pltpu
from jax.experimental.pallas import tpu_sc as plsc

T, U, I = 8, 1024, 2048
B = 4096
TOP_K = 50
NW = 32
NQ = 4
BQ = B // NQ
PER_W = BQ // NW
CHUNK = 16
NCHUNK = PER_W // CHUNK

_R = 512
_G = B // _R


def _f32_key(np_val):
    b = np.array(np_val, np.float32).view(np.int32)
    return int(b) if b >= 0 else int(b ^ 0x7FFFFFFF)


_LO_KEY = _f32_key(-1.5)
_HI_KEY = _f32_key(1.5)


_TB = 512


def _tr_body(in_ref, out_ref):
    out_ref[...] = in_ref[0].T


def _transpose_qos(qos):
    return pl.pallas_call(
        _tr_body,
        grid=(T, I // _TB),
        in_specs=[pl.BlockSpec((1, U, _TB), lambda t, ib: (t, 0, ib))],
        out_specs=pl.BlockSpec((_TB, U), lambda t, ib: (t * (I // _TB) + ib, 0)),
        out_shape=jax.ShapeDtypeStruct((T * I, U), jnp.float32),
    )(qos)


def _sc_body(qt, usim, tid, iid, uid, out_col, out_sim,
             tball, iball, uball, rall,
             colb0, colb1, simb0, simb1,
             sgc0, sgc1, sgs0, sgs1, swc0, swc1, sws0, sws1):
    cid = lax.axis_index("c")
    sid = lax.axis_index("s")
    wid = sid * 2 + cid
    w0 = wid * PER_W

    pltpu.sync_copy(tid.at[pl.ds(w0, PER_W)], tball)
    pltpu.sync_copy(iid.at[pl.ds(w0, PER_W)], iball)
    pltpu.sync_copy(uid.at[pl.ds(w0, PER_W)], uball)
    for k in range(PER_W // 16):
        sl = pl.ds(k * 16, 16)
        rall[sl] = tball[sl] * I + iball[sl]

    colb = (colb0, colb1)
    simb = (simb0, simb1)
    sgc = (sgc0, sgc1)
    sgs = (sgs0, sgs1)
    swc = (swc0, swc1)
    sws = (sws0, sws1)

    gc = [None, None]
    gs = [None, None]
    wc = [None, None]
    ws = [None, None]

    for ch in range(NCHUNK):
        s = ch % 2
        if ch >= 2:
            wc[s].wait()
            ws[s].wait()
        isl = pl.ds(ch * CHUNK, CHUNK)
        gc[s] = pltpu.async_copy(qt.at[rall.at[isl]], colb[s], sgc[s])
        gs[s] = pltpu.async_copy(usim.at[uball.at[isl]], simb[s], sgs[s])
        if ch >= 1:
            p = (ch - 1) % 2
            gc[p].wait()
            gs[p].wait()
            b0 = w0 + (ch - 1) * CHUNK
            wc[p] = pltpu.async_copy(colb[p], out_col.at[pl.ds(b0, CHUNK)], swc[p])
            ws[p] = pltpu.async_copy(simb[p], out_sim.at[pl.ds(b0, CHUNK)], sws[p])

    last = NCHUNK - 1
    s = last % 2
    gc[s].wait()
    gs[s].wait()
    b0 = w0 + last * CHUNK
    wc[s] = pltpu.async_copy(colb[s], out_col.at[pl.ds(b0, CHUNK)], swc[s])
    ws[s] = pltpu.async_copy(simb[s], out_sim.at[pl.ds(b0, CHUNK)], sws[s])
    wc[0].wait()
    ws[0].wait()
    wc[1].wait()
    ws[1].wait()


@functools.cache
def _get_sc_gather():
    return pl.kernel(
        _sc_body,
        mesh=plsc.VectorSubcoreMesh(core_axis_name="c", subcore_axis_name="s"),
        out_type=[
            jax.ShapeDtypeStruct((BQ, U), jnp.float32),
            jax.ShapeDtypeStruct((BQ, U), jnp.float32),
        ],
        scratch_types=[
            pltpu.VMEM((PER_W,), jnp.int32),
            pltpu.VMEM((PER_W,), jnp.int32),
            pltpu.VMEM((PER_W,), jnp.int32),
            pltpu.VMEM((PER_W,), jnp.int32),
            pltpu.VMEM((CHUNK, U), jnp.float32),
            pltpu.VMEM((CHUNK, U), jnp.float32),
            pltpu.VMEM((CHUNK, U), jnp.float32),
            pltpu.VMEM((CHUNK, U), jnp.float32),
            pltpu.SemaphoreType.DMA,
            pltpu.SemaphoreType.DMA,
            pltpu.SemaphoreType.DMA,
            pltpu.SemaphoreType.DMA,
            pltpu.SemaphoreType.DMA,
            pltpu.SemaphoreType.DMA,
            pltpu.SemaphoreType.DMA,
            pltpu.SemaphoreType.DMA,
        ],
    )


def _tc_body(tid_ref, uid_ref, col_ref, sim_ref, uavg_ref, out_ref):
    col = col_ref[...]
    sim = sim_ref[...]
    msim = jnp.where(col > 0.0, sim, 0.0)

    kbits = lax.bitcast_convert_type(msim, jnp.int32)
    key = jnp.where(kbits >= 0, kbits, kbits ^ jnp.int32(0x7FFFFFFF))

    tcol = tid_ref[:, 0:1]
    ucol = uid_ref[:, 0:1]

    avgv = jnp.zeros((_R, U), jnp.float32)
    for t in range(T):
        avgv = jnp.where(tcol == t, uavg_ref[t:t + 1, :], avgv)

    def _cnt(m):
        return jnp.sum((key > m).astype(jnp.int32), axis=1, keepdims=True)

    def bs(i, m):
        m2 = m + (jnp.int32(1) << (jnp.int32(30) - i))
        return jnp.where(_cnt(m2) >= TOP_K, m2, m)

    m0 = jnp.full((_R, 1), _LO_KEY, jnp.int32)
    thk = lax.fori_loop(0, 31, bs, m0) + 1

    tbits = jnp.where(thk >= 0, thk, thk ^ jnp.int32(0x7FFFFFFF))
    theta = lax.bitcast_convert_type(tbits, jnp.float32)

    strict = key > thk
    tie = key == thk
    d = col - avgv
    zero = jnp.zeros((), jnp.float32)
    s1 = jnp.sum(jnp.where(strict, msim, zero), axis=1, keepdims=True)
    c = jnp.sum(jnp.where(strict, 1.0, zero), axis=1, keepdims=True)
    nums = jnp.sum(jnp.where(strict, msim * d, zero), axis=1, keepdims=True)
    t2 = jnp.sum(jnp.where(tie, d, zero), axis=1, keepdims=True)
    tcnt = jnp.sum(jnp.where(tie, 1.0, zero), axis=1, keepdims=True)

    rem = jnp.float32(TOP_K) - c
    num = nums + theta * (rem / tcnt) * t2
    den = s1 + theta * rem

    lane = lax.broadcasted_iota(jnp.int32, (_R, U), 1)
    avg_u = jnp.sum(jnp.where(lane == ucol, avgv, zero), axis=1, keepdims=True)

    pred = avg_u + num / (den + 1e-8)
    out_ref[...] = jnp.broadcast_to(pred, (_R, 128))


def _tc_call(tid2, uid2, col, sim, uavg, interpret=False):
    return pl.pallas_call(
        _tc_body,
        grid=(col.shape[0] // _R,),
        in_specs=[
            pl.BlockSpec((_R, 128), lambda g: (g, 0)),
            pl.BlockSpec((_R, 128), lambda g: (g, 0)),
            pl.BlockSpec((_R, U), lambda g: (g, 0)),
            pl.BlockSpec((_R, U), lambda g: (g, 0)),
            pl.BlockSpec((T, U), lambda g: (0, 0)),
        ],
        out_specs=pl.BlockSpec((_R, 128), lambda g: (g, 0)),
        out_shape=jax.ShapeDtypeStruct((col.shape[0], 128), jnp.float32),
        interpret=interpret,
    )(tid2, uid2, col, sim, uavg)


def kernel(qos, user_avg, user_sim_agg, user_id, item_id, time_id):
    qt = _transpose_qos(qos)
    sc = _get_sc_gather()
    tid2 = jnp.broadcast_to(time_id[:, None], (B, 128))
    uid2 = jnp.broadcast_to(user_id[:, None], (B, 128))
    gathered = [sc(qt, user_sim_agg,
                   time_id[q * BQ:(q + 1) * BQ],
                   item_id[q * BQ:(q + 1) * BQ],
                   user_id[q * BQ:(q + 1) * BQ]) for q in range(NQ)]
    preds = [_tc_call(tid2[q * BQ:(q + 1) * BQ], uid2[q * BQ:(q + 1) * BQ],
                      col, sim, user_avg)
             for q, (col, sim) in enumerate(gathered)]
    return jnp.concatenate(preds, axis=0)[:, 0]
